# baseline (speedup 1.0000x reference)
import functools

import jax
import jax.numpy as jnp
from jax import lax
from jax.experimental import pallas as pl
from jax.experimental.pallas import tpu as pltpu
from jax.experimental.pallas import tpu_sc as plsc

N = 10000
D = 128
E = 320000
NC = 2
NS = 16
NW = NC * NS
K = 128
CH = 80
CHP = 16
NPASS = CH // CHP
EPAD = NW * CH * K
RPT = 632
NPAD = NS * RPT

_mesh = plsc.VectorSubcoreMesh(core_axis_name="c", subcore_axis_name="s")


NBLK = NPAD // 128
NFULL = N // 128
NTAIL = N - NFULL * 128


@functools.partial(
    pl.kernel,
    mesh=_mesh,
    out_type=jax.ShapeDtypeStruct((NW, NBLK, 128), jnp.float32),
    scratch_types=[
        pltpu.VMEM((CH, K), jnp.int32),
        pltpu.VMEM((NPAD,), jnp.float32),
    ],
    compiler_params=pltpu.CompilerParams(needs_layout_passes=False),
)
def _deg_kernel(dst_hbm, out_hbm, dst_v, hist_v):
    cid = lax.axis_index("c")
    sid = lax.axis_index("s")
    wid = cid * NS + sid
    pltpu.sync_copy(dst_hbm.at[wid], dst_v)
    zeros = jnp.zeros((16,), jnp.float32)

    def zbody(j, c):
        hist_v[pl.ds(j * 16, 16)] = zeros
        return c

    lax.fori_loop(0, NPAD // 16, zbody, 0)
    ones = jnp.ones((16,), jnp.float32)

    def body(j, c):
        for kk in range(K // 16):
            idx = dst_v[j, pl.ds(kk * 16, 16)]
            plsc.addupdate_scatter(hist_v, [idx], ones)
        return c

    lax.fori_loop(0, CH, body, 0)

    def wbody(j, c):
        pltpu.sync_copy(hist_v.at[pl.ds(j * 128, 128)], out_hbm.at[wid, j])
        return c

    lax.fori_loop(0, NBLK, wbody, 0)


@functools.partial(
    pl.kernel,
    mesh=_mesh,
    out_type=jax.ShapeDtypeStruct((NC, NPAD, D), jnp.float32),
    scratch_types=[
        pltpu.VMEM((CH, K), jnp.int32),
        pltpu.VMEM((CH, K), jnp.int32),
        pltpu.VMEM((K, D), jnp.float32),
        pltpu.VMEM_SHARED((NPAD, D), jnp.float32),
        pltpu.SemaphoreType.DMA,
    ],
)
def _agg_kernel(src_hbm, dst_hbm, g_hbm, zero_hbm, out_hbm,
                src_v, dst_v, buf, agg_sh, sem):
    cid = lax.axis_index("c")
    sid = lax.axis_index("s")
    wid = cid * NS + sid
    pltpu.sync_copy(src_hbm.at[wid], src_v)
    pltpu.sync_copy(dst_hbm.at[wid], dst_v)
    row0 = sid * RPT
    pltpu.sync_copy(zero_hbm.at[pl.ds(row0, RPT)], agg_sh.at[pl.ds(row0, RPT)])
    plsc.subcore_barrier()

    def body(j, carry):
        pltpu.async_copy(g_hbm.at[src_v.at[j]], buf, sem).wait()
        pltpu.sync_copy(buf, agg_sh.at[dst_v.at[j]], add=True)
        return carry

    lax.fori_loop(0, CH, body, 0)
    plsc.subcore_barrier()
    pltpu.sync_copy(agg_sh.at[pl.ds(row0, RPT)],
                    out_hbm.at[cid, pl.ds(row0, RPT)])


def _dis_t(d2):
    deg = jnp.sum(d2, axis=0) + 1.0
    return jnp.transpose(lax.rsqrt(deg))


def _pre_body(x_ref, w_ref, d2_ref, g_ref):
    dis_t = _dis_t(d2_ref[...])
    h = jnp.dot(x_ref[...], w_ref[...], preferred_element_type=jnp.float32)
    for c in range(NFULL):
        g_ref[c * 128:(c + 1) * 128, :] = (
            h[c * 128:(c + 1) * 128, :] * dis_t[:, c:c + 1])
    g_ref[NFULL * 128:N, :] = (
        h[NFULL * 128:N, :] * dis_t[:NTAIL, NFULL:NFULL + 1])


_pre_tc = pl.pallas_call(
    _pre_body, out_shape=jax.ShapeDtypeStruct((N, D), jnp.float32))

_SQRT_HALF = 0.7071067811865476


def _post_body(p_ref, g_ref, x_ref, d2_ref, b_ref, gam_ref, bet_ref, y_ref):
    dis_t = _dis_t(d2_ref[...])
    for c in range(NFULL):
        sl = slice(c * 128, (c + 1) * 128)
        t = p_ref[0, sl, :] + p_ref[1, sl, :] + g_ref[sl, :]
        y_ref[sl, :] = t * dis_t[:, c:c + 1] + b_ref[...]
    sl = slice(NFULL * 128, N)
    t = p_ref[0, sl, :] + p_ref[1, sl, :] + g_ref[sl, :]
    y_ref[sl, :] = t * dis_t[:NTAIL, NFULL:NFULL + 1] + b_ref[...]
    gcn = y_ref[...]
    mu = jnp.mean(gcn, axis=0, keepdims=True)
    cen = gcn - mu
    var = jnp.mean(cen * cen, axis=0, keepdims=True)
    bn = gam_ref[...] * cen * lax.rsqrt(var + 1e-5) + bet_ref[...]
    z = bn + x_ref[...]
    y_ref[...] = 0.5 * z * (1.0 + lax.erf(z * _SQRT_HALF))


_post_tc = pl.pallas_call(
    _post_body, out_shape=jax.ShapeDtypeStruct((N, D), jnp.float32))


def kernel(x, edge_index, W, b, gamma, beta):
    src = edge_index[0].astype(jnp.int32)
    dst = edge_index[1].astype(jnp.int32)
    pad = EPAD - E
    srcp = jnp.concatenate([src, jnp.zeros((pad,), jnp.int32)]).reshape(NW, CH, K)
    dstp = jnp.concatenate([dst, jnp.full((pad,), N, jnp.int32)]).reshape(NW, CH, K)
    zerosD = jnp.zeros((NPAD, D), jnp.float32)

    d2 = _deg_kernel(dstp)
    g = _pre_tc(x, W, d2)
    p = _agg_kernel(srcp, dstp, g, zerosD)
    return _post_tc(p, g, x, d2,
                    b.reshape(1, D), gamma.reshape(1, D), beta.reshape(1, D))


# speedup vs baseline: 2.2562x; 2.2562x over previous
import functools

import jax
import jax.numpy as jnp
from jax import lax
from jax.experimental import pallas as pl
from jax.experimental.pallas import tpu as pltpu
from jax.experimental.pallas import tpu_sc as plsc

N = 10000
D = 128
E = 320000
NC = 2
NS = 16
NW = NC * NS
K = 128
CH = 79
EPT = E // NW
PPT = CH * K - EPT
EPAD = NW * CH * K
RPT = 632
NPAD = NS * RPT

_mesh = plsc.VectorSubcoreMesh(core_axis_name="c", subcore_axis_name="s")


NBLK = NPAD // 128
NFULL = N // 128
NTAIL = N - NFULL * 128


@functools.partial(
    pl.kernel,
    mesh=_mesh,
    out_type=jax.ShapeDtypeStruct((NW, NBLK, 128), jnp.float32),
    scratch_types=[
        pltpu.VMEM((CH, K), jnp.int32),
        pltpu.VMEM((NPAD,), jnp.float32),
    ],
    compiler_params=pltpu.CompilerParams(needs_layout_passes=False),
)
def _deg_kernel(dst_hbm, out_hbm, dst_v, hist_v):
    cid = lax.axis_index("c")
    sid = lax.axis_index("s")
    wid = cid * NS + sid
    pltpu.sync_copy(dst_hbm.at[wid], dst_v)
    zeros = jnp.zeros((16,), jnp.float32)

    def zbody(j, c):
        hist_v[pl.ds(j * 16, 16)] = zeros
        return c

    lax.fori_loop(0, NPAD // 16, zbody, 0)
    ones = jnp.ones((16,), jnp.float32)

    def body(j, c):
        for kk in range(K // 16):
            idx = dst_v[j, pl.ds(kk * 16, 16)]
            plsc.addupdate_scatter(hist_v, [idx], ones)
        return c

    lax.fori_loop(0, CH, body, 0)

    def wbody(j, c):
        pltpu.sync_copy(hist_v.at[pl.ds(j * 128, 128)], out_hbm.at[wid, j])
        return c

    lax.fori_loop(0, NBLK, wbody, 0)


@functools.partial(
    pl.kernel,
    mesh=_mesh,
    out_type=jax.ShapeDtypeStruct((NC, NPAD, D), jnp.float32),
    scratch_types=[
        pltpu.VMEM((CH, K), jnp.int32),
        pltpu.VMEM((CH, K), jnp.int32),
        pltpu.VMEM((K, D), jnp.float32),
        pltpu.VMEM_SHARED((NPAD, D), jnp.float32),
        pltpu.SemaphoreType.DMA,
    ],
)
def _agg_kernel(src_hbm, dst_hbm, g_hbm, zero_hbm, out_hbm,
                src_v, dst_v, buf, agg_sh, sem):
    cid = lax.axis_index("c")
    sid = lax.axis_index("s")
    wid = cid * NS + sid
    pltpu.sync_copy(src_hbm.at[wid], src_v)
    pltpu.sync_copy(dst_hbm.at[wid], dst_v)
    row0 = sid * RPT
    pltpu.sync_copy(zero_hbm.at[pl.ds(row0, RPT)], agg_sh.at[pl.ds(row0, RPT)])
    plsc.subcore_barrier()

    def body(j, carry):
        pltpu.async_copy(g_hbm.at[src_v.at[j]], buf, sem).wait()
        pltpu.sync_copy(buf, agg_sh.at[dst_v.at[j]], add=True)
        return carry

    lax.fori_loop(0, CH, body, 0)
    plsc.subcore_barrier()
    pltpu.sync_copy(agg_sh.at[pl.ds(row0, RPT)],
                    out_hbm.at[cid, pl.ds(row0, RPT)])


def _dis_t(d2):
    deg = jnp.sum(d2, axis=0) + 1.0
    return jnp.transpose(lax.rsqrt(deg))


def _pre_body(x_ref, w_ref, d2_ref, g_ref):
    dis_t = _dis_t(d2_ref[...])
    h = jnp.dot(x_ref[...], w_ref[...], preferred_element_type=jnp.float32)
    for c in range(NFULL):
        g_ref[c * 128:(c + 1) * 128, :] = (
            h[c * 128:(c + 1) * 128, :] * dis_t[:, c:c + 1])
    g_ref[NFULL * 128:N, :] = (
        h[NFULL * 128:N, :] * dis_t[:NTAIL, NFULL:NFULL + 1])


_pre_tc = pl.pallas_call(
    _pre_body, out_shape=jax.ShapeDtypeStruct((N, D), jnp.float32))

_SQRT_HALF = 0.7071067811865476


def _post_body(p_ref, g_ref, x_ref, d2_ref, b_ref, gam_ref, bet_ref, y_ref):
    dis_t = _dis_t(d2_ref[...])
    for c in range(NFULL):
        sl = slice(c * 128, (c + 1) * 128)
        t = p_ref[0, sl, :] + p_ref[1, sl, :] + g_ref[sl, :]
        y_ref[sl, :] = t * dis_t[:, c:c + 1] + b_ref[...]
    sl = slice(NFULL * 128, N)
    t = p_ref[0, sl, :] + p_ref[1, sl, :] + g_ref[sl, :]
    y_ref[sl, :] = t * dis_t[:NTAIL, NFULL:NFULL + 1] + b_ref[...]
    gcn = y_ref[...]
    mu = jnp.mean(gcn, axis=0, keepdims=True)
    cen = gcn - mu
    var = jnp.mean(cen * cen, axis=0, keepdims=True)
    bn = gam_ref[...] * cen * lax.rsqrt(var + 1e-5) + bet_ref[...]
    z = bn + x_ref[...]
    y_ref[...] = 0.5 * z * (1.0 + lax.erf(z * _SQRT_HALF))


_post_tc = pl.pallas_call(
    _post_body, out_shape=jax.ShapeDtypeStruct((N, D), jnp.float32))


def kernel(x, edge_index, W, b, gamma, beta):
    src = edge_index[0].astype(jnp.int32)
    dst = edge_index[1].astype(jnp.int32)
    pad_src = jnp.broadcast_to(jnp.arange(PPT, dtype=jnp.int32), (NW, PPT))
    pad_dst = jnp.broadcast_to(N + jnp.arange(PPT, dtype=jnp.int32), (NW, PPT))
    srcp = jnp.concatenate([src.reshape(NW, EPT), pad_src], axis=1
                           ).reshape(NW, CH, K)
    dstp = jnp.concatenate([dst.reshape(NW, EPT), pad_dst], axis=1
                           ).reshape(NW, CH, K)
    zerosD = jnp.zeros((NPAD, D), jnp.float32)

    d2 = _deg_kernel(dstp)
    g = _pre_tc(x, W, d2)
    p = _agg_kernel(srcp, dstp, g, zerosD)
    return _post_tc(p, g, x, d2,
                    b.reshape(1, D), gamma.reshape(1, D), beta.reshape(1, D))


# speedup vs baseline: 2.4217x; 1.0733x over previous
import functools

import jax
import jax.numpy as jnp
from jax import lax
from jax.experimental import pallas as pl
from jax.experimental.pallas import tpu as pltpu
from jax.experimental.pallas import tpu_sc as plsc

N = 10000
D = 128
E = 320000
NC = 2
NS = 16
NW = NC * NS
K = 128
CH = 80
CHP = 16
NPASS = CH // CHP
EPT = E // NW
PPT = CH * K - EPT
EPAD = NW * CH * K
RPT = 640
NPAD = NS * RPT

_mesh = plsc.VectorSubcoreMesh(core_axis_name="c", subcore_axis_name="s")


NBLK = NPAD // 128
NFULL = N // 128
NTAIL = N - NFULL * 128


@functools.partial(
    pl.kernel,
    mesh=_mesh,
    out_type=jax.ShapeDtypeStruct((NW, NBLK, 128), jnp.float32),
    scratch_types=[
        pltpu.VMEM((CH, K), jnp.int32),
        pltpu.VMEM((NPAD,), jnp.float32),
    ],
    compiler_params=pltpu.CompilerParams(needs_layout_passes=False),
)
def _deg_kernel(dst_hbm, out_hbm, dst_v, hist_v):
    cid = lax.axis_index("c")
    sid = lax.axis_index("s")
    wid = cid * NS + sid
    pltpu.sync_copy(dst_hbm.at[wid], dst_v)
    zeros = jnp.zeros((16,), jnp.float32)

    def zbody(j, c):
        hist_v[pl.ds(j * 16, 16)] = zeros
        return c

    lax.fori_loop(0, NPAD // 16, zbody, 0)
    ones = jnp.ones((16,), jnp.float32)

    def body(j, c):
        for kk in range(K // 16):
            idx = dst_v[j, pl.ds(kk * 16, 16)]
            plsc.addupdate_scatter(hist_v, [idx], ones)
        return c

    lax.fori_loop(0, CH, body, 0)

    def wbody(j, c):
        pltpu.sync_copy(hist_v.at[pl.ds(j * 128, 128)], out_hbm.at[wid, j])
        return c

    lax.fori_loop(0, NBLK, wbody, 0)


@functools.partial(
    pl.kernel,
    mesh=_mesh,
    out_type=jax.ShapeDtypeStruct((NC, NPAD, D), jnp.float32),
    scratch_types=[
        pltpu.VMEM((CHP, K), jnp.int32),
        pltpu.VMEM((CHP, K), jnp.int32),
        pltpu.VMEM((K, D), jnp.float32),
        pltpu.VMEM((K, D), jnp.float32),
        pltpu.VMEM_SHARED((NPAD, D), jnp.float32),
        pltpu.SemaphoreType.DMA,
        pltpu.SemaphoreType.DMA,
    ],
)
def _agg_kernel(src_hbm, dst_hbm, g_hbm, zero_hbm, out_hbm,
                src_v, dst_v, buf0, buf1, agg_sh, sem0, sem1):
    cid = lax.axis_index("c")
    sid = lax.axis_index("s")
    wid = cid * NS + sid
    row0 = sid * RPT
    pltpu.sync_copy(zero_hbm.at[pl.ds(row0, RPT)], agg_sh.at[pl.ds(row0, RPT)])
    plsc.subcore_barrier()

    for p in range(NPASS):
        pltpu.sync_copy(src_hbm.at[wid, pl.ds(p * CHP, CHP)], src_v)
        pltpu.sync_copy(dst_hbm.at[wid, pl.ds(p * CHP, CHP)], dst_v)

        def body(jj, carry):
            j0 = jj * 2
            j1 = j0 + 1
            h0 = pltpu.async_copy(g_hbm.at[src_v.at[j0]], buf0, sem0)
            h1 = pltpu.async_copy(g_hbm.at[src_v.at[j1]], buf1, sem1)
            h0.wait()
            pltpu.sync_copy(buf0, agg_sh.at[dst_v.at[j0]], add=True)
            h1.wait()
            pltpu.sync_copy(buf1, agg_sh.at[dst_v.at[j1]], add=True)
            return carry

        lax.fori_loop(0, CHP // 2, body, 0)
    plsc.subcore_barrier()
    pltpu.sync_copy(agg_sh.at[pl.ds(row0, RPT)],
                    out_hbm.at[cid, pl.ds(row0, RPT)])


def _dis_t(d2):
    deg = jnp.sum(d2, axis=0) + 1.0
    return jnp.transpose(lax.rsqrt(deg))


def _pre_body(x_ref, w_ref, d2_ref, g_ref):
    dis_t = _dis_t(d2_ref[...])
    h = jnp.dot(x_ref[...], w_ref[...], preferred_element_type=jnp.float32)
    for c in range(NFULL):
        g_ref[c * 128:(c + 1) * 128, :] = (
            h[c * 128:(c + 1) * 128, :] * dis_t[:, c:c + 1])
    g_ref[NFULL * 128:N, :] = (
        h[NFULL * 128:N, :] * dis_t[:NTAIL, NFULL:NFULL + 1])


_pre_tc = pl.pallas_call(
    _pre_body, out_shape=jax.ShapeDtypeStruct((N, D), jnp.float32))

_SQRT_HALF = 0.7071067811865476


def _post_body(p_ref, g_ref, x_ref, d2_ref, b_ref, gam_ref, bet_ref, y_ref):
    dis_t = _dis_t(d2_ref[...])
    for c in range(NFULL):
        sl = slice(c * 128, (c + 1) * 128)
        t = p_ref[0, sl, :] + p_ref[1, sl, :] + g_ref[sl, :]
        y_ref[sl, :] = t * dis_t[:, c:c + 1] + b_ref[...]
    sl = slice(NFULL * 128, N)
    t = p_ref[0, sl, :] + p_ref[1, sl, :] + g_ref[sl, :]
    y_ref[sl, :] = t * dis_t[:NTAIL, NFULL:NFULL + 1] + b_ref[...]
    gcn = y_ref[...]
    mu = jnp.mean(gcn, axis=0, keepdims=True)
    cen = gcn - mu
    var = jnp.mean(cen * cen, axis=0, keepdims=True)
    bn = gam_ref[...] * cen * lax.rsqrt(var + 1e-5) + bet_ref[...]
    z = bn + x_ref[...]
    y_ref[...] = 0.5 * z * (1.0 + lax.erf(z * _SQRT_HALF))


_post_tc = pl.pallas_call(
    _post_body, out_shape=jax.ShapeDtypeStruct((N, D), jnp.float32))


def kernel(x, edge_index, W, b, gamma, beta):
    src = edge_index[0].astype(jnp.int32)
    dst = edge_index[1].astype(jnp.int32)
    pad_src = jnp.broadcast_to(jnp.arange(PPT, dtype=jnp.int32) % K, (NW, PPT))
    pad_dst = jnp.broadcast_to(N + jnp.arange(PPT, dtype=jnp.int32) % K,
                               (NW, PPT))
    srcp = jnp.concatenate([src.reshape(NW, EPT), pad_src], axis=1
                           ).reshape(NW, CH, K)
    dstp = jnp.concatenate([dst.reshape(NW, EPT), pad_dst], axis=1
                           ).reshape(NW, CH, K)
    zerosD = jnp.zeros((NPAD, D), jnp.float32)

    d2 = _deg_kernel(dstp)
    g = _pre_tc(x, W, d2)
    p = _agg_kernel(srcp, dstp, g, zerosD)
    return _post_tc(p, g, x, d2,
                    b.reshape(1, D), gamma.reshape(1, D), beta.reshape(1, D))


# speedup vs baseline: 2.4469x; 1.0104x over previous
import functools

import jax
import jax.numpy as jnp
from jax import lax
from jax.experimental import pallas as pl
from jax.experimental.pallas import tpu as pltpu
from jax.experimental.pallas import tpu_sc as plsc

N = 10000
D = 128
E = 320000
NC = 2
NS = 16
NW = NC * NS
K = 128
CH = 80
CHP = 16
NPASS = CH // CHP
EPT = E // NW
PPT = CH * K - EPT
EPAD = NW * CH * K
RPT = 640
NPAD = NS * RPT

_mesh = plsc.VectorSubcoreMesh(core_axis_name="c", subcore_axis_name="s")


NBLK = NPAD // 128
NFULL = N // 128
NTAIL = N - NFULL * 128


@functools.partial(
    pl.kernel,
    mesh=_mesh,
    out_type=jax.ShapeDtypeStruct((NW, NBLK, 128), jnp.float32),
    scratch_types=[
        pltpu.VMEM((CH, K), jnp.int32),
        pltpu.VMEM((NPAD,), jnp.float32),
    ],
    compiler_params=pltpu.CompilerParams(needs_layout_passes=False),
)
def _deg_kernel(dst_hbm, out_hbm, dst_v, hist_v):
    cid = lax.axis_index("c")
    sid = lax.axis_index("s")
    wid = cid * NS + sid
    pltpu.sync_copy(dst_hbm.at[wid], dst_v)
    zeros = jnp.zeros((16,), jnp.float32)

    def zbody(j, c):
        hist_v[pl.ds(j * 16, 16)] = zeros
        return c

    lax.fori_loop(0, NPAD // 16, zbody, 0)
    ones = jnp.ones((16,), jnp.float32)

    def body(j, c):
        for kk in range(K // 16):
            idx = dst_v[j, pl.ds(kk * 16, 16)]
            plsc.addupdate_scatter(hist_v, [idx], ones)
        return c

    lax.fori_loop(0, CH, body, 0)

    def wbody(j, c):
        pltpu.sync_copy(hist_v.at[pl.ds(j * 128, 128)], out_hbm.at[wid, j])
        return c

    lax.fori_loop(0, NBLK, wbody, 0)


@functools.partial(
    pl.kernel,
    mesh=_mesh,
    out_type=jax.ShapeDtypeStruct((NC, NPAD, D), jnp.float32),
    scratch_types=[
        pltpu.VMEM((CHP, K), jnp.int32),
        pltpu.VMEM((CHP, K), jnp.int32),
        pltpu.VMEM((K, D), jnp.float32),
        pltpu.VMEM((K, D), jnp.float32),
        pltpu.VMEM_SHARED((NPAD, D), jnp.float32),
        pltpu.SemaphoreType.DMA,
        pltpu.SemaphoreType.DMA,
    ],
)
def _agg_kernel(src_hbm, dst_hbm, g_hbm, zero_hbm, out_hbm,
                src_v, dst_v, buf0, buf1, agg_sh, sem0, sem1):
    cid = lax.axis_index("c")
    sid = lax.axis_index("s")
    wid = cid * NS + sid
    row0 = sid * RPT

    @pl.when(cid == 0)
    def _():
        pltpu.sync_copy(g_hbm.at[pl.ds(row0, RPT)],
                        agg_sh.at[pl.ds(row0, RPT)])

    @pl.when(cid != 0)
    def _():
        pltpu.sync_copy(zero_hbm.at[pl.ds(row0, RPT)],
                        agg_sh.at[pl.ds(row0, RPT)])

    plsc.subcore_barrier()

    for p in range(NPASS):
        pltpu.sync_copy(src_hbm.at[wid, pl.ds(p * CHP, CHP)], src_v)
        pltpu.sync_copy(dst_hbm.at[wid, pl.ds(p * CHP, CHP)], dst_v)

        def body(jj, carry):
            j0 = jj * 2
            j1 = j0 + 1
            h0 = pltpu.async_copy(g_hbm.at[src_v.at[j0]], buf0, sem0)
            h1 = pltpu.async_copy(g_hbm.at[src_v.at[j1]], buf1, sem1)
            h0.wait()
            pltpu.sync_copy(buf0, agg_sh.at[dst_v.at[j0]], add=True)
            h1.wait()
            pltpu.sync_copy(buf1, agg_sh.at[dst_v.at[j1]], add=True)
            return carry

        lax.fori_loop(0, CHP // 2, body, 0)
    plsc.subcore_barrier()
    pltpu.sync_copy(agg_sh.at[pl.ds(row0, RPT)],
                    out_hbm.at[cid, pl.ds(row0, RPT)])


def _dis_t(d2):
    deg = jnp.sum(d2, axis=0) + 1.0
    return jnp.transpose(lax.rsqrt(deg))


def _mm_body(x_ref, w_ref, h_ref):
    h_ref[...] = jnp.dot(x_ref[...], w_ref[...],
                         preferred_element_type=jnp.float32)


_mm_tc = pl.pallas_call(
    _mm_body, out_shape=jax.ShapeDtypeStruct((N, D), jnp.float32))


def _scale_body(h_ref, d2_ref, g_ref):
    dis_t = _dis_t(d2_ref[...])
    for c in range(NFULL):
        g_ref[c * 128:(c + 1) * 128, :] = (
            h_ref[c * 128:(c + 1) * 128, :] * dis_t[:, c:c + 1])
    g_ref[NFULL * 128:N, :] = (
        h_ref[NFULL * 128:N, :] * dis_t[:NTAIL, NFULL:NFULL + 1])
    g_ref[N:, :] = jnp.zeros((NPAD - N, D), jnp.float32)


_scale_tc = pl.pallas_call(
    _scale_body, out_shape=jax.ShapeDtypeStruct((NPAD, D), jnp.float32))

_SQRT_HALF = 0.7071067811865476


def _post_body(p_ref, x_ref, d2_ref, b_ref, gam_ref, bet_ref, y_ref):
    dis_t = _dis_t(d2_ref[...])
    for c in range(NFULL):
        sl = slice(c * 128, (c + 1) * 128)
        t = p_ref[0, sl, :] + p_ref[1, sl, :]
        y_ref[sl, :] = t * dis_t[:, c:c + 1] + b_ref[...]
    sl = slice(NFULL * 128, N)
    t = p_ref[0, sl, :] + p_ref[1, sl, :]
    y_ref[sl, :] = t * dis_t[:NTAIL, NFULL:NFULL + 1] + b_ref[...]
    gcn = y_ref[...]
    mu = jnp.mean(gcn, axis=0, keepdims=True)
    cen = gcn - mu
    var = jnp.mean(cen * cen, axis=0, keepdims=True)
    bn = gam_ref[...] * cen * lax.rsqrt(var + 1e-5) + bet_ref[...]
    z = bn + x_ref[...]
    y_ref[...] = 0.5 * z * (1.0 + lax.erf(z * _SQRT_HALF))


_post_tc = pl.pallas_call(
    _post_body, out_shape=jax.ShapeDtypeStruct((N, D), jnp.float32))


def kernel(x, edge_index, W, b, gamma, beta):
    src = edge_index[0].astype(jnp.int32)
    dst = edge_index[1].astype(jnp.int32)
    pad_src = jnp.broadcast_to(jnp.arange(PPT, dtype=jnp.int32) % K, (NW, PPT))
    pad_dst = jnp.broadcast_to(N + jnp.arange(PPT, dtype=jnp.int32) % K,
                               (NW, PPT))
    srcp = jnp.concatenate([src.reshape(NW, EPT), pad_src], axis=1
                           ).reshape(NW, CH, K)
    dstp = jnp.concatenate([dst.reshape(NW, EPT), pad_dst], axis=1
                           ).reshape(NW, CH, K)
    zerosD = jnp.zeros((NPAD, D), jnp.float32)

    d2 = _deg_kernel(dstp)
    h = _mm_tc(x, W)
    g = _scale_tc(h, d2)
    p = _agg_kernel(srcp, dstp, g, zerosD)
    return _post_tc(p, x, d2,
                    b.reshape(1, D), gamma.reshape(1, D), beta.reshape(1, D))
